# baseline (speedup 1.0000x reference)
import functools
import jax
import jax.numpy as jnp
from jax.experimental import pallas as pl
from jax.experimental.pallas import tpu as pltpu
from jax.experimental.pallas import tpu_sc as plsc

_BN = 4
_LANES = 16


def kernel(node_input, affine_weight, expand_index):
    n, m, c = node_input.shape
    num_l = affine_weight.shape[1]
    aw2 = affine_weight.reshape(num_l, c)
    ei = expand_index.astype(jnp.int32)
    kc = c // _LANES

    mesh = plsc.VectorSubcoreMesh(core_axis_name="core", subcore_axis_name="subcore")

    @functools.partial(
        pl.kernel,
        out_type=jax.ShapeDtypeStruct((n, m, c), jnp.float32),
        mesh=mesh,
        scratch_types=[
            pltpu.VMEM((m,), jnp.int32),
            pltpu.VMEM((m, c), jnp.float32),
            pltpu.SemaphoreType.DMA,
        ],
    )
    def _sc(x_hbm, aw_hbm, ei_hbm, o_hbm, ei_v, w_v, sem):
        pltpu.async_copy(ei_hbm, ei_v, sem).wait()
        pltpu.async_copy(aw_hbm.at[ei_v], w_v, sem).wait()

        def body(in_v, out_v):
            @pl.loop(0, _BN)
            def _(nn):
                for mm in range(m):
                    for k in range(kc):
                        sl = pl.ds(k * _LANES, _LANES)
                        out_v[nn, mm, sl] = in_v[nn, mm, sl] * w_v[mm, sl]

        pltpu.emit_pipeline(
            body,
            grid=(n // _BN,),
            in_specs=[pl.BlockSpec((_BN, m, c), lambda i: (i, 0, 0))],
            out_specs=[pl.BlockSpec((_BN, m, c), lambda i: (i, 0, 0))],
            core_axis_name=("core", "subcore"),
            dimension_semantics=(pltpu.PARALLEL,),
        )(x_hbm, o_hbm)

    return _sc(node_input, aw2, ei)
